# baseline (speedup 1.0000x reference)
import functools

import jax
import jax.numpy as jnp
from jax import lax
from jax.experimental import pallas as pl
from jax.experimental.pallas import tpu as pltpu
from jax.experimental.pallas import tpu_sc as plsc

B = 16384
N_CONT = 13
N_CAT = 26
CARD = 100000
EMB = 16
NX = N_CONT + N_CAT
NTAB = N_CAT * CARD

L = 16
NC = 2
NS = 16
NW = NC * NS
RPW = B // NW
NBLK = RPW // L
NH = RPW // 128

CBLK = 98304
NF1 = 17
NB1 = 18
SB2 = 17
NB2 = 10
SOFF2 = SB2 * CBLK


def _tc1_body(tt_ref, wt_ref, xt_ref, s_ref, xcm_ref):
    pid = pl.program_id(0)
    col0 = pid * CBLK
    f0 = col0 // CARD
    fi = lax.broadcasted_iota(jnp.int32, (2, N_CAT), 1)
    tgt = f0 + lax.broadcasted_iota(jnp.int32, (2, N_CAT), 0)
    oh = (fi == tgt).astype(jnp.float32)
    w01 = jax.lax.dot_general(
        oh, wt_ref[:], (((1,), (1,)), ((), ())),
        preferred_element_type=jnp.float32,
    )
    r = jax.lax.dot_general(
        w01, tt_ref[:], (((1,), (0,)), ((), ())),
        preferred_element_type=jnp.float32,
    )
    colrel = lax.broadcasted_iota(jnp.int32, (1, CBLK), 1)
    use1 = colrel >= (f0 + 1) * CARD - col0
    s_ref[:] = jnp.where(use1, r[1:2, :], r[0:1, :])[0]

    @pl.when(pid == 0)
    def _():
        for c in range(NX):
            xcm_ref[pl.ds(c * B, B)] = xt_ref[c, :]


_tc_scale_1 = pl.pallas_call(
    _tc1_body,
    grid=(NB1,),
    in_specs=[
        pl.BlockSpec((EMB, CBLK), lambda i: (0, i)),
        pl.BlockSpec((EMB, N_CAT), lambda i: (0, 0)),
        pl.BlockSpec((NX, B), lambda i: (0, 0)),
    ],
    out_specs=[
        pl.BlockSpec((CBLK,), lambda i: (i,)),
        pl.BlockSpec((NX * B,), lambda i: (0,)),
    ],
    out_shape=[
        jax.ShapeDtypeStruct((NB1 * CBLK,), jnp.float32),
        jax.ShapeDtypeStruct((NX * B,), jnp.float32),
    ],
    compiler_params=pltpu.CompilerParams(
        dimension_semantics=("arbitrary",),
    ),
)


def _tc2_body(tt_ref, wt_ref, s_ref):
    col0 = (pl.program_id(0) + SB2) * CBLK
    f0 = col0 // CARD
    fi = lax.broadcasted_iota(jnp.int32, (2, N_CAT), 1)
    tgt = f0 + lax.broadcasted_iota(jnp.int32, (2, N_CAT), 0)
    oh = (fi == tgt).astype(jnp.float32)
    w01 = jax.lax.dot_general(
        oh, wt_ref[:], (((1,), (1,)), ((), ())),
        preferred_element_type=jnp.float32,
    )
    r = jax.lax.dot_general(
        w01, tt_ref[:], (((1,), (0,)), ((), ())),
        preferred_element_type=jnp.float32,
    )
    colrel = lax.broadcasted_iota(jnp.int32, (1, CBLK), 1)
    use1 = colrel >= (f0 + 1) * CARD - col0
    s_ref[:] = jnp.where(use1, r[1:2, :], r[0:1, :])[0]


_tc_scale_2 = pl.pallas_call(
    _tc2_body,
    grid=(NB2,),
    in_specs=[
        pl.BlockSpec((EMB, CBLK), lambda i: (0, i + SB2)),
        pl.BlockSpec((EMB, N_CAT), lambda i: (0, 0)),
    ],
    out_specs=pl.BlockSpec((CBLK,), lambda i: (i,)),
    out_shape=jax.ShapeDtypeStruct((NB2 * CBLK,), jnp.float32),
    compiler_params=pltpu.CompilerParams(
        dimension_semantics=("arbitrary",),
    ),
)

_mesh = plsc.VectorSubcoreMesh(core_axis_name="c", subcore_axis_name="s")
_sc_params = pltpu.CompilerParams(
    needs_layout_passes=False, use_tc_tiling_on_sc=False
)


NF2 = N_CAT - NF1


@functools.partial(
    pl.kernel,
    mesh=_mesh,
    compiler_params=_sc_params,
    out_type=[
        jax.ShapeDtypeStruct((B,), jnp.float32),
        jax.ShapeDtypeStruct((NW, NF2, NH, 128), jnp.int32),
    ],
    scratch_types=[
        pltpu.VMEM((NX, RPW), jnp.float32),
        pltpu.VMEM((NF1, NH, 128), jnp.int32),
        pltpu.VMEM((NF2, NH, 128), jnp.int32),
        pltpu.VMEM((NF1, NH, 128), jnp.float32),
        pltpu.VMEM((2, L), jnp.float32),
        pltpu.VMEM((RPW,), jnp.float32),
        pltpu.SemaphoreType.DMA,
        pltpu.SemaphoreType.DMA,
    ],
)
def _sc_logits_a(x_hbm, s_hbm, w_hbm, out_hbm, idxb_hbm,
                 xcv, idxv, idxbv, sv, wv, outv, sem, xsem):
    wid = lax.axis_index("s") * NC + lax.axis_index("c")
    wbase = wid * RPW

    for c in range(NX):
        pltpu.async_copy(x_hbm.at[pl.ds(c * B + wbase, RPW)], xcv.at[c], xsem)
    pltpu.sync_copy(w_hbm, wv)
    for c in range(NX):
        pltpu.make_async_copy(
            x_hbm.at[pl.ds(c * B + wbase, RPW)], xcv.at[c], xsem
        ).wait()

    def field_body(j, _):
        base = j * CARD
        crow = N_CONT + j
        for t in range(NBLK):
            vals = xcv[crow, pl.ds(t * L, L)]
            fi = vals.astype(jnp.int32) + base
            idxv[j, t // 8, pl.ds((t % 8) * L, L)] = fi
        for h in range(NH):
            pltpu.async_copy(s_hbm.at[idxv.at[j, h]], sv.at[j, h], sem)
        return _

    lax.fori_loop(0, NF1, field_body, None)

    def bfield_body(j, _):
        base = j * CARD + (NF1 * CARD - SOFF2)
        crow = N_CONT + NF1 + j
        for t in range(NBLK):
            vals = xcv[crow, pl.ds(t * L, L)]
            idxbv[j, t // 8, pl.ds((t % 8) * L, L)] = (
                vals.astype(jnp.int32) + base
            )
        return _

    lax.fori_loop(0, NF2, bfield_body, None)
    pltpu.async_copy(idxbv, idxb_hbm.at[wid], xsem)

    def drain_body(j, _):
        for h in range(NH):
            pltpu.make_async_copy(
                s_hbm.at[idxv.at[j, h]], sv.at[j, h], sem
            ).wait()
        return _

    lax.fori_loop(0, NF1, drain_body, None)

    wrow = wv[0, :]
    wcont = [wrow[c] for c in range(N_CONT)]
    bvec = wv[1, :]

    def block_body(t, _):
        h = t // 8
        off = (t % 8) * L
        acc = bvec
        for c in range(N_CONT):
            acc = acc + xcv[c, pl.ds(t * L, L)] * wcont[c]
        for j in range(NF1):
            acc = acc + sv[j, h, pl.ds(off, L)]
        outv[pl.ds(t * L, L)] = acc
        return _

    lax.fori_loop(0, NBLK, block_body, None)
    pltpu.sync_copy(outv, out_hbm.at[pl.ds(wbase, RPW)])
    pltpu.make_async_copy(idxbv, idxb_hbm.at[wid], xsem).wait()


@functools.partial(
    pl.kernel,
    mesh=_mesh,
    compiler_params=_sc_params,
    out_type=jax.ShapeDtypeStruct((B,), jnp.float32),
    scratch_types=[
        pltpu.VMEM((NF2, NH, 128), jnp.int32),
        pltpu.VMEM((NF2, NH, 128), jnp.float32),
        pltpu.VMEM((RPW,), jnp.float32),
        pltpu.VMEM((RPW,), jnp.float32),
        pltpu.SemaphoreType.DMA,
    ],
)
def _sc_logits_b(idxb_hbm, s_hbm, p_hbm, out_hbm, idxv, sv, pv, outv, sem):
    wid = lax.axis_index("s") * NC + lax.axis_index("c")
    wbase = wid * RPW

    pltpu.sync_copy(idxb_hbm.at[wid], idxv)

    def field_body(j, _):
        for h in range(NH):
            pltpu.async_copy(s_hbm.at[idxv.at[j, h]], sv.at[j, h], sem)
        return _

    lax.fori_loop(0, NF2, field_body, None)
    pltpu.sync_copy(p_hbm.at[pl.ds(wbase, RPW)], pv)

    def drain_body(j, _):
        for h in range(NH):
            pltpu.make_async_copy(
                s_hbm.at[idxv.at[j, h]], sv.at[j, h], sem
            ).wait()
        return _

    lax.fori_loop(0, NF2, drain_body, None)

    def block_body(t, _):
        h = t // 8
        off = (t % 8) * L
        acc = pv[pl.ds(t * L, L)]
        for j in range(NF2):
            acc = acc + sv[j, h, pl.ds(off, L)]
        outv[pl.ds(t * L, L)] = acc
        return _

    lax.fori_loop(0, NBLK, block_body, None)
    pltpu.sync_copy(outv, out_hbm.at[pl.ds(wbase, RPW)])


def kernel(x, table, W, b):
    wt = W[N_CONT:, 0].reshape(N_CAT, EMB).T
    wcont = jnp.pad(W[:N_CONT, 0], (0, L - N_CONT))
    brow = jnp.broadcast_to(b.reshape(1, 1), (1, L))
    wv = jnp.concatenate([wcont[None, :], brow], axis=0)
    tt = table.T
    s_lo, xcm = _tc_scale_1(tt, wt, x.T)
    partial, idxb = _sc_logits_a(xcm, s_lo, wv)
    s_hi = _tc_scale_2(tt, wt)
    out = _sc_logits_b(idxb, s_hi, partial)
    return out.reshape(B, 1)


# speedup vs baseline: 1.0030x; 1.0030x over previous
import functools

import jax
import jax.numpy as jnp
from jax import lax
from jax.experimental import pallas as pl
from jax.experimental.pallas import tpu as pltpu
from jax.experimental.pallas import tpu_sc as plsc

B = 16384
N_CONT = 13
N_CAT = 26
CARD = 100000
EMB = 16
NX = N_CONT + N_CAT
NTAB = N_CAT * CARD

L = 16
NC = 2
NS = 16
NW = NC * NS
RPW = B // NW
NBLK = RPW // L
NH = RPW // 128

CBLK = 98304
NF1 = 16
NB1 = 17
SB2 = 16
NB2 = 11
SOFF2 = SB2 * CBLK


def _tc1_body(tt_ref, wt_ref, xt_ref, s_ref, xcm_ref):
    pid = pl.program_id(0)
    col0 = pid * CBLK
    f0 = col0 // CARD
    fi = lax.broadcasted_iota(jnp.int32, (2, N_CAT), 1)
    tgt = f0 + lax.broadcasted_iota(jnp.int32, (2, N_CAT), 0)
    oh = (fi == tgt).astype(jnp.float32)
    w01 = jax.lax.dot_general(
        oh, wt_ref[:], (((1,), (1,)), ((), ())),
        preferred_element_type=jnp.float32,
    )
    r = jax.lax.dot_general(
        w01, tt_ref[:], (((1,), (0,)), ((), ())),
        preferred_element_type=jnp.float32,
    )
    colrel = lax.broadcasted_iota(jnp.int32, (1, CBLK), 1)
    use1 = colrel >= (f0 + 1) * CARD - col0
    s_ref[:] = jnp.where(use1, r[1:2, :], r[0:1, :])[0]

    @pl.when(pid == 0)
    def _():
        for c in range(NX):
            xcm_ref[pl.ds(c * B, B)] = xt_ref[c, :]


_tc_scale_1 = pl.pallas_call(
    _tc1_body,
    grid=(NB1,),
    in_specs=[
        pl.BlockSpec((EMB, CBLK), lambda i: (0, i)),
        pl.BlockSpec((EMB, N_CAT), lambda i: (0, 0)),
        pl.BlockSpec((NX, B), lambda i: (0, 0)),
    ],
    out_specs=[
        pl.BlockSpec((CBLK,), lambda i: (i,)),
        pl.BlockSpec((NX * B,), lambda i: (0,)),
    ],
    out_shape=[
        jax.ShapeDtypeStruct((NB1 * CBLK,), jnp.float32),
        jax.ShapeDtypeStruct((NX * B,), jnp.float32),
    ],
    compiler_params=pltpu.CompilerParams(
        dimension_semantics=("arbitrary",),
    ),
)


def _tc2_body(tt_ref, wt_ref, s_ref):
    col0 = (pl.program_id(0) + SB2) * CBLK
    f0 = col0 // CARD
    fi = lax.broadcasted_iota(jnp.int32, (2, N_CAT), 1)
    tgt = f0 + lax.broadcasted_iota(jnp.int32, (2, N_CAT), 0)
    oh = (fi == tgt).astype(jnp.float32)
    w01 = jax.lax.dot_general(
        oh, wt_ref[:], (((1,), (1,)), ((), ())),
        preferred_element_type=jnp.float32,
    )
    r = jax.lax.dot_general(
        w01, tt_ref[:], (((1,), (0,)), ((), ())),
        preferred_element_type=jnp.float32,
    )
    colrel = lax.broadcasted_iota(jnp.int32, (1, CBLK), 1)
    use1 = colrel >= (f0 + 1) * CARD - col0
    s_ref[:] = jnp.where(use1, r[1:2, :], r[0:1, :])[0]


_tc_scale_2 = pl.pallas_call(
    _tc2_body,
    grid=(NB2,),
    in_specs=[
        pl.BlockSpec((EMB, CBLK), lambda i: (0, i + SB2)),
        pl.BlockSpec((EMB, N_CAT), lambda i: (0, 0)),
    ],
    out_specs=pl.BlockSpec((CBLK,), lambda i: (i,)),
    out_shape=jax.ShapeDtypeStruct((NB2 * CBLK,), jnp.float32),
    compiler_params=pltpu.CompilerParams(
        dimension_semantics=("arbitrary",),
    ),
)

_mesh = plsc.VectorSubcoreMesh(core_axis_name="c", subcore_axis_name="s")
_sc_params = pltpu.CompilerParams(
    needs_layout_passes=False, use_tc_tiling_on_sc=False
)


NF2 = N_CAT - NF1


@functools.partial(
    pl.kernel,
    mesh=_mesh,
    compiler_params=_sc_params,
    out_type=[
        jax.ShapeDtypeStruct((B,), jnp.float32),
        jax.ShapeDtypeStruct((NW, NF2, NH, 128), jnp.int32),
    ],
    scratch_types=[
        pltpu.VMEM((NX, RPW), jnp.float32),
        pltpu.VMEM((NF1, NH, 128), jnp.int32),
        pltpu.VMEM((NF2, NH, 128), jnp.int32),
        pltpu.VMEM((NF1, NH, 128), jnp.float32),
        pltpu.VMEM((2, L), jnp.float32),
        pltpu.VMEM((RPW,), jnp.float32),
        pltpu.SemaphoreType.DMA,
        pltpu.SemaphoreType.DMA,
    ],
)
def _sc_logits_a(x_hbm, s_hbm, w_hbm, out_hbm, idxb_hbm,
                 xcv, idxv, idxbv, sv, wv, outv, sem, xsem):
    wid = lax.axis_index("s") * NC + lax.axis_index("c")
    wbase = wid * RPW

    for c in range(NX):
        pltpu.async_copy(x_hbm.at[pl.ds(c * B + wbase, RPW)], xcv.at[c], xsem)
    pltpu.sync_copy(w_hbm, wv)
    for c in range(NX):
        pltpu.make_async_copy(
            x_hbm.at[pl.ds(c * B + wbase, RPW)], xcv.at[c], xsem
        ).wait()

    def field_body(j, _):
        base = j * CARD
        crow = N_CONT + j
        for t in range(NBLK):
            vals = xcv[crow, pl.ds(t * L, L)]
            fi = vals.astype(jnp.int32) + base
            idxv[j, t // 8, pl.ds((t % 8) * L, L)] = fi
        for h in range(NH):
            pltpu.async_copy(s_hbm.at[idxv.at[j, h]], sv.at[j, h], sem)
        return _

    lax.fori_loop(0, NF1, field_body, None)

    def bfield_body(j, _):
        base = j * CARD + (NF1 * CARD - SOFF2)
        crow = N_CONT + NF1 + j
        for t in range(NBLK):
            vals = xcv[crow, pl.ds(t * L, L)]
            idxbv[j, t // 8, pl.ds((t % 8) * L, L)] = (
                vals.astype(jnp.int32) + base
            )
        return _

    lax.fori_loop(0, NF2, bfield_body, None)
    pltpu.async_copy(idxbv, idxb_hbm.at[wid], xsem)

    def drain_body(j, _):
        for h in range(NH):
            pltpu.make_async_copy(
                s_hbm.at[idxv.at[j, h]], sv.at[j, h], sem
            ).wait()
        return _

    lax.fori_loop(0, NF1, drain_body, None)

    wrow = wv[0, :]
    wcont = [wrow[c] for c in range(N_CONT)]
    bvec = wv[1, :]

    def block_body(t, _):
        h = t // 8
        off = (t % 8) * L
        acc = bvec
        for c in range(N_CONT):
            acc = acc + xcv[c, pl.ds(t * L, L)] * wcont[c]
        for j in range(NF1):
            acc = acc + sv[j, h, pl.ds(off, L)]
        outv[pl.ds(t * L, L)] = acc
        return _

    lax.fori_loop(0, NBLK, block_body, None)
    pltpu.sync_copy(outv, out_hbm.at[pl.ds(wbase, RPW)])
    pltpu.make_async_copy(idxbv, idxb_hbm.at[wid], xsem).wait()


@functools.partial(
    pl.kernel,
    mesh=_mesh,
    compiler_params=_sc_params,
    out_type=jax.ShapeDtypeStruct((B,), jnp.float32),
    scratch_types=[
        pltpu.VMEM((NF2, NH, 128), jnp.int32),
        pltpu.VMEM((NF2, NH, 128), jnp.float32),
        pltpu.VMEM((RPW,), jnp.float32),
        pltpu.VMEM((RPW,), jnp.float32),
        pltpu.SemaphoreType.DMA,
    ],
)
def _sc_logits_b(idxb_hbm, s_hbm, p_hbm, out_hbm, idxv, sv, pv, outv, sem):
    wid = lax.axis_index("s") * NC + lax.axis_index("c")
    wbase = wid * RPW

    pltpu.sync_copy(idxb_hbm.at[wid], idxv)

    def field_body(j, _):
        for h in range(NH):
            pltpu.async_copy(s_hbm.at[idxv.at[j, h]], sv.at[j, h], sem)
        return _

    lax.fori_loop(0, NF2, field_body, None)
    pltpu.sync_copy(p_hbm.at[pl.ds(wbase, RPW)], pv)

    def drain_body(j, _):
        for h in range(NH):
            pltpu.make_async_copy(
                s_hbm.at[idxv.at[j, h]], sv.at[j, h], sem
            ).wait()
        return _

    lax.fori_loop(0, NF2, drain_body, None)

    def block_body(t, _):
        h = t // 8
        off = (t % 8) * L
        acc = pv[pl.ds(t * L, L)]
        for j in range(NF2):
            acc = acc + sv[j, h, pl.ds(off, L)]
        outv[pl.ds(t * L, L)] = acc
        return _

    lax.fori_loop(0, NBLK, block_body, None)
    pltpu.sync_copy(outv, out_hbm.at[pl.ds(wbase, RPW)])


def kernel(x, table, W, b):
    wt = W[N_CONT:, 0].reshape(N_CAT, EMB).T
    wcont = jnp.pad(W[:N_CONT, 0], (0, L - N_CONT))
    brow = jnp.broadcast_to(b.reshape(1, 1), (1, L))
    wv = jnp.concatenate([wcont[None, :], brow], axis=0)
    tt = table.T
    s_lo, xcm = _tc_scale_1(tt, wt, x.T)
    partial, idxb = _sc_logits_a(xcm, s_lo, wv)
    s_hi = _tc_scale_2(tt, wt)
    out = _sc_logits_b(idxb, s_hi, partial)
    return out.reshape(B, 1)
